# baseline (speedup 1.0000x reference)
import functools

import jax
import jax.numpy as jnp
from jax import lax
from jax.experimental import pallas as pl
from jax.experimental.pallas import tpu as pltpu
from jax.experimental.pallas import tpu_sc as plsc

NUM_FIELDS = 26
VOCAB = 100000
HIDDEN = 32
BATCH = 16384

_P = NUM_FIELDS * HIDDEN
_SPLIT = 2
_PH = _P // _SPLIT
_OCH = 4096


def _sc_info():
    try:
        info = plsc.get_sparse_core_info()
        return info.num_cores, info.num_subcores
    except Exception:
        return 2, 16


@functools.lru_cache(maxsize=None)
def _make_gather(nc, ns, p_base):
    nw = nc * ns
    pairs_w = _PH // nw
    n_och = BATCH // _OCH

    mesh = plsc.VectorSubcoreMesh(core_axis_name="c", subcore_axis_name="s")

    @functools.partial(
        pl.kernel,
        out_type=jax.ShapeDtypeStruct((_PH, BATCH), jnp.float32),
        mesh=mesh,
        scratch_types=[
            pltpu.VMEM((VOCAB,), jnp.float32),
            pltpu.VMEM((BATCH,), jnp.int32),
            pltpu.VMEM((2, _OCH), jnp.float32),
            pltpu.SemaphoreType.DMA,
            pltpu.SemaphoreType.DMA,
            pltpu.SemaphoreType.DMA,
        ],
        compiler_params=pltpu.CompilerParams(
            needs_layout_passes=False, disable_bounds_checks=True),
    )
    def gather_k(tt_hbm, xt_hbm, out_hbm, row_v, xv, obuf, sem_row, sem_x,
                 sem_out):
        w = lax.axis_index("s") * nc + lax.axis_index("c")
        p0 = w * pairs_w
        rot = w % pairs_w

        pending = [None, None]
        f_prev = jnp.int32(-1)
        for i in range(pairs_w):
            p = p0 + (i + rot) % pairs_w
            f = (p_base + p) // HIDDEN

            @pl.when(f != f_prev)
            def _():
                pltpu.async_copy(xt_hbm.at[f], xv, sem_x)
            row_cp = pltpu.async_copy(tt_hbm.at[p_base + p], row_v, sem_row)
            @pl.when(f != f_prev)
            def _():
                pltpu.make_async_copy(xt_hbm.at[f], xv, sem_x).wait()
            row_cp.wait()
            f_prev = f

            for c in range(n_och):
                slot = c % 2
                if pending[slot] is not None:
                    pending[slot].wait()

                @plsc.parallel_loop(0, _OCH, 16, unroll=8)
                def _(j):
                    idx = xv[pl.ds(c * _OCH + j, 16)]
                    obuf[slot, pl.ds(j, 16)] = plsc.load_gather(
                        row_v, [idx])

                pending[slot] = pltpu.async_copy(
                    obuf.at[slot],
                    out_hbm.at[p, pl.ds(c * _OCH, _OCH)],
                    sem_out,
                )
        for cp in pending:
            if cp is not None:
                cp.wait()

    return gather_k


def _mm_acc_body(ct_ref, w_ref, acc_ref, o_ref):
    o_ref[...] = (
        lax.dot_general(
            ct_ref[...], w_ref[...],
            dimension_numbers=(((0,), (0,)), ((), ())),
            preferred_element_type=jnp.float32,
        )
        + acc_ref[...]
    )


def _matmul_acc(catT, Wh, acc):
    bm = 2048
    return pl.pallas_call(
        _mm_acc_body,
        grid=(BATCH // bm,),
        in_specs=[
            pl.BlockSpec((_PH, bm), lambda i: (0, i)),
            pl.BlockSpec((_PH, HIDDEN), lambda i: (0, 0)),
            pl.BlockSpec((bm, HIDDEN), lambda i: (i, 0)),
        ],
        out_specs=pl.BlockSpec((bm, HIDDEN), lambda i: (i, 0)),
        out_shape=jax.ShapeDtypeStruct((BATCH, HIDDEN), jnp.float32),
    )(catT, Wh, acc)


def kernel(x, tables, W, b):
    nc, ns = _sc_info()
    tt = jnp.transpose(tables, (0, 2, 1)).reshape(_P, VOCAB)
    xt = x.T
    acc = jnp.broadcast_to(b.reshape(1, HIDDEN), (BATCH, HIDDEN))
    for s in range(_SPLIT):
        catT = _make_gather(nc, ns, s * _PH)(tt, xt)
        acc = _matmul_acc(catT, W[s * _PH:(s + 1) * _PH], acc)
    return acc


# speedup vs baseline: 1.0668x; 1.0668x over previous
import functools

import jax
import jax.numpy as jnp
from jax import lax
from jax.experimental import pallas as pl
from jax.experimental.pallas import tpu as pltpu
from jax.experimental.pallas import tpu_sc as plsc

NUM_FIELDS = 26
VOCAB = 100000
HIDDEN = 32
BATCH = 16384

_P = NUM_FIELDS * HIDDEN
_OCH = 4096
_NSTREAM = 4
_QL = 25088


def _sc_info():
    try:
        info = plsc.get_sparse_core_info()
        return info.num_cores, info.num_subcores
    except Exception:
        return 2, 16


@functools.lru_cache(maxsize=None)
def _make_gather(nc, ns):
    nw = nc * ns
    pairs_w = _P // nw
    n_och = BATCH // _OCH

    mesh = plsc.VectorSubcoreMesh(core_axis_name="c", subcore_axis_name="s")

    @functools.partial(
        pl.kernel,
        out_type=jax.ShapeDtypeStruct((_P, BATCH), jnp.float32),
        mesh=mesh,
        scratch_types=[
            pltpu.VMEM((VOCAB,), jnp.float32),
            pltpu.VMEM((BATCH,), jnp.int32),
            pltpu.VMEM((2, _OCH), jnp.float32),
            pltpu.SemaphoreType.DMA,
            pltpu.SemaphoreType.DMA,
            pltpu.SemaphoreType.DMA,
        ],
        compiler_params=pltpu.CompilerParams(
            needs_layout_passes=False, disable_bounds_checks=True),
    )
    def gather_k(tt_hbm, xt_hbm, out_hbm, row_v, xv, obuf, sem_row, sem_x,
                 sem_out):
        w = lax.axis_index("s") * nc + lax.axis_index("c")
        p0 = w * pairs_w
        rot = w % pairs_w

        pending = [None, None]
        f_prev = jnp.int32(-1)
        for i in range(pairs_w):
            p = p0 + (i + rot) % pairs_w
            f = p // HIDDEN

            @pl.when(f != f_prev)
            def _():
                pltpu.async_copy(xt_hbm.at[f], xv, sem_x)
            row_cps = [pltpu.async_copy(tt_hbm.at[p], row_v, sem_row)]
            @pl.when(f != f_prev)
            def _():
                pltpu.make_async_copy(xt_hbm.at[f], xv, sem_x).wait()
            for cp in row_cps:
                cp.wait()
            f_prev = f

            for c in range(n_och):
                slot = c % 2
                if pending[slot] is not None:
                    pending[slot].wait()

                @plsc.parallel_loop(0, _OCH, 16, unroll=8)
                def _(j):
                    idx = xv[pl.ds(c * _OCH + j, 16)]
                    obuf[slot, pl.ds(j, 16)] = plsc.load_gather(
                        row_v, [idx])

                pending[slot] = pltpu.async_copy(
                    obuf.at[slot],
                    out_hbm.at[p, pl.ds(c * _OCH, _OCH)],
                    sem_out,
                )
        for cp in pending:
            if cp is not None:
                cp.wait()

    return gather_k


def _mm_body(ct_ref, w_ref, b_ref, o_ref):
    o_ref[...] = (
        lax.dot_general(
            ct_ref[...].astype(jnp.bfloat16), w_ref[...].astype(jnp.bfloat16),
            dimension_numbers=(((0,), (0,)), ((), ())),
            preferred_element_type=jnp.float32,
        )
        + b_ref[...]
    )


def _matmul(catT, W, b2d):
    bm = 2048
    return pl.pallas_call(
        _mm_body,
        grid=(BATCH // bm,),
        in_specs=[
            pl.BlockSpec((_P, bm), lambda i: (0, i)),
            pl.BlockSpec((_P, HIDDEN), lambda i: (0, 0)),
            pl.BlockSpec((1, HIDDEN), lambda i: (0, 0)),
        ],
        out_specs=pl.BlockSpec((bm, HIDDEN), lambda i: (i, 0)),
        out_shape=jax.ShapeDtypeStruct((BATCH, HIDDEN), jnp.float32),
    )(catT, W, b2d)


def kernel(x, tables, W, b):
    nc, ns = _sc_info()
    tt = jnp.transpose(tables, (0, 2, 1)).reshape(_P, VOCAB)
    xt = x.T
    catT = _make_gather(nc, ns)(tt, xt)
    return _matmul(catT, W, b.reshape(1, HIDDEN))


# speedup vs baseline: 1.1105x; 1.0410x over previous
import functools

import jax
import jax.numpy as jnp
from jax import lax
from jax.experimental import pallas as pl
from jax.experimental.pallas import tpu as pltpu
from jax.experimental.pallas import tpu_sc as plsc

NUM_FIELDS = 26
VOCAB = 100000
HIDDEN = 32
BATCH = 16384

_P = NUM_FIELDS * HIDDEN
_OCH = 4096


def _sc_info():
    try:
        info = plsc.get_sparse_core_info()
        return info.num_cores, info.num_subcores
    except Exception:
        return 2, 16


@functools.lru_cache(maxsize=None)
def _make_gather(nc, ns):
    nw = nc * ns
    pairs_w = _P // nw
    n_och = BATCH // _OCH

    mesh = plsc.VectorSubcoreMesh(core_axis_name="c", subcore_axis_name="s")

    @functools.partial(
        pl.kernel,
        out_type=jax.ShapeDtypeStruct((_P, BATCH), jnp.float32),
        mesh=mesh,
        scratch_types=[
            pltpu.VMEM((VOCAB,), jnp.float32),
            pltpu.VMEM((BATCH,), jnp.int32),
            pltpu.VMEM((2, _OCH), jnp.float32),
            pltpu.SemaphoreType.DMA,
            pltpu.SemaphoreType.DMA,
            pltpu.SemaphoreType.DMA,
        ],
        compiler_params=pltpu.CompilerParams(
            needs_layout_passes=False, disable_bounds_checks=True),
    )
    def gather_k(tt_hbm, xt_hbm, out_hbm, row_v, xv, obuf, sem_row, sem_x,
                 sem_out):
        w = lax.axis_index("s") * nc + lax.axis_index("c")
        p0 = w * pairs_w
        rot = w % pairs_w

        pending = [None, None]
        f_prev = jnp.int32(-1)
        for i in range(pairs_w):
            p = p0 + (i + rot) % pairs_w
            f = p // HIDDEN

            @pl.when(f != f_prev)
            def _():
                pltpu.async_copy(xt_hbm.at[f], xv, sem_x)
            row_cps = [pltpu.async_copy(tt_hbm.at[p], row_v, sem_row)]
            @pl.when(f != f_prev)
            def _():
                pltpu.make_async_copy(xt_hbm.at[f], xv, sem_x).wait()
            for cp in row_cps:
                cp.wait()
            f_prev = f

            for c in range(n_och):
                slot = c % 2
                if pending[slot] is not None:
                    pending[slot].wait()

                @plsc.parallel_loop(0, _OCH, 16, unroll=8)
                def _(j):
                    idx = xv[pl.ds(c * _OCH + j, 16)]
                    obuf[slot, pl.ds(j, 16)] = plsc.load_gather(
                        row_v, [idx])

                pending[slot] = pltpu.async_copy(
                    obuf.at[slot],
                    out_hbm.at[p, pl.ds(c * _OCH, _OCH)],
                    sem_out,
                )
        for cp in pending:
            if cp is not None:
                cp.wait()

    return gather_k


def _mm_body(ct_ref, w_ref, b_ref, o_ref):
    o_ref[...] = (
        lax.dot_general(
            w_ref[...], ct_ref[...],
            dimension_numbers=(((0,), (0,)), ((), ())),
            preferred_element_type=jnp.float32,
        )
        + b_ref[...]
    )


def _matmul(catT, W, bcol):
    bm = 2048
    return pl.pallas_call(
        _mm_body,
        grid=(BATCH // bm,),
        in_specs=[
            pl.BlockSpec((_P, bm), lambda i: (0, i)),
            pl.BlockSpec((_P, HIDDEN), lambda i: (0, 0)),
            pl.BlockSpec((HIDDEN, 1), lambda i: (0, 0)),
        ],
        out_specs=pl.BlockSpec((HIDDEN, bm), lambda i: (0, i)),
        out_shape=jax.ShapeDtypeStruct((HIDDEN, BATCH), jnp.float32),
    )(catT, W, bcol)


def kernel(x, tables, W, b):
    nc, ns = _sc_info()
    tt = jnp.transpose(tables, (0, 2, 1)).reshape(_P, VOCAB)
    xt = x.T
    catT = _make_gather(nc, ns)(tt, xt)
    return _matmul(catT, W, b.reshape(HIDDEN, 1)).T


# speedup vs baseline: 1.1128x; 1.0020x over previous
import functools

import jax
import jax.numpy as jnp
from jax import lax
from jax.experimental import pallas as pl
from jax.experimental.pallas import tpu as pltpu
from jax.experimental.pallas import tpu_sc as plsc

NUM_FIELDS = 26
VOCAB = 100000
HIDDEN = 32
BATCH = 16384

_P = NUM_FIELDS * HIDDEN
_OCH = 4096


def _sc_info():
    try:
        info = plsc.get_sparse_core_info()
        return info.num_cores, info.num_subcores
    except Exception:
        return 2, 16


@functools.lru_cache(maxsize=None)
def _make_gather(nc, ns):
    nw = nc * ns
    pairs_w = _P // nw
    n_och = BATCH // _OCH

    mesh = plsc.VectorSubcoreMesh(core_axis_name="c", subcore_axis_name="s")

    @functools.partial(
        pl.kernel,
        out_type=jax.ShapeDtypeStruct((_P, BATCH), jnp.float32),
        mesh=mesh,
        scratch_types=[
            pltpu.VMEM((VOCAB,), jnp.float32),
            pltpu.VMEM((BATCH,), jnp.int32),
            pltpu.VMEM((2, _OCH), jnp.float32),
            pltpu.SemaphoreType.DMA,
            pltpu.SemaphoreType.DMA,
            pltpu.SemaphoreType.DMA,
        ],
        compiler_params=pltpu.CompilerParams(
            needs_layout_passes=False, disable_bounds_checks=True),
    )
    def gather_k(tt_hbm, xt_hbm, out_hbm, row_v, xv, obuf, sem_row, sem_x,
                 sem_out):
        w = lax.axis_index("s") * nc + lax.axis_index("c")
        p0 = w * pairs_w
        rot = w % pairs_w

        pending = [None, None]
        f_prev = jnp.int32(-1)
        for i in range(pairs_w):
            p = p0 + (i + rot) % pairs_w
            f = p // HIDDEN

            @pl.when(f != f_prev)
            def _():
                pltpu.async_copy(xt_hbm.at[f], xv, sem_x)
            row_cp = pltpu.async_copy(tt_hbm.at[p], row_v, sem_row)
            @pl.when(f != f_prev)
            def _():
                pltpu.make_async_copy(xt_hbm.at[f], xv, sem_x).wait()
            row_cp.wait()
            f_prev = f

            for c in range(n_och):
                slot = c % 2
                if pending[slot] is not None:
                    pending[slot].wait()

                @plsc.parallel_loop(0, _OCH, 16, unroll=8)
                def _(j):
                    idx = xv[pl.ds(c * _OCH + j, 16)]
                    obuf[slot, pl.ds(j, 16)] = plsc.load_gather(
                        row_v, [idx])

                pending[slot] = pltpu.async_copy(
                    obuf.at[slot],
                    out_hbm.at[p, pl.ds(c * _OCH, _OCH)],
                    sem_out,
                )
        for cp in pending:
            if cp is not None:
                cp.wait()

    return gather_k


def _mm_body(ct_ref, w_ref, b_ref, o_ref):
    o_ref[...] = (
        lax.dot_general(
            w_ref[...], ct_ref[...],
            dimension_numbers=(((0,), (0,)), ((), ())),
            preferred_element_type=jnp.float32,
        )
        + b_ref[...]
    )


def _matmul(catT, W, bcol):
    bm = 2048
    return pl.pallas_call(
        _mm_body,
        grid=(BATCH // bm,),
        in_specs=[
            pl.BlockSpec((_P, bm), lambda i: (0, i)),
            pl.BlockSpec((_P, HIDDEN), lambda i: (0, 0)),
            pl.BlockSpec((HIDDEN, 1), lambda i: (0, 0)),
        ],
        out_specs=pl.BlockSpec((HIDDEN, bm), lambda i: (0, i)),
        out_shape=jax.ShapeDtypeStruct((HIDDEN, BATCH), jnp.float32),
    )(catT, W, bcol)


def kernel(x, tables, W, b):
    nc, ns = _sc_info()
    tt = jnp.transpose(tables, (0, 2, 1)).reshape(_P, VOCAB)
    xt = x.T
    catT = _make_gather(nc, ns)(tt, xt)
    return _matmul(catT, W, b.reshape(HIDDEN, 1)).T
